# baseline (speedup 1.0000x reference)
import functools

import jax
import jax.numpy as jnp
from jax import lax
from jax.experimental import pallas as pl
from jax.experimental.pallas import tpu as pltpu
from jax.experimental.pallas import tpu_sc as plsc

_NC = 2
_NS = 16
_NW = _NC * _NS
_BLK = 128

MAXN = 3
NSHF = 16
NRBF = 48
CUT3 = 4.0
OUT_STD = 1.0
OUT_MEAN = 0.0


def _sigmoid(v):
    return jax.nn.sigmoid(v)


def _silu(v):
    return v * jax.nn.sigmoid(v)


def _embed_body(types_ref, table_ref, out_ref):
    t = types_ref[...]
    ntypes = table_ref.shape[0]
    iota = jax.lax.broadcasted_iota(jnp.int32, (t.shape[0], ntypes), 1)
    onehot = (t == iota).astype(jnp.float32)
    out_ref[...] = jnp.dot(onehot, table_ref[...],
                           preferred_element_type=jnp.float32)


def _embed(atom_types, table, ch=2000):
    n = atom_types.shape[0]
    ch = ch if n % ch == 0 else n
    u = table.shape[1]
    t2 = atom_types.reshape(n, 1).astype(jnp.int32)
    return pl.pallas_call(
        _embed_body,
        grid=(n // ch,),
        in_specs=[
            pl.BlockSpec((ch, 1), lambda i: (i, 0)),
            pl.BlockSpec(table.shape, lambda i: (0, 0)),
        ],
        out_specs=pl.BlockSpec((ch, u), lambda i: (i, 0)),
        out_shape=jax.ShapeDtypeStruct((n, u), jnp.float32),
    )(t2, table)


def _bondproj_body(b0_ref, b1_ref, b2_ref, w_ref, bias_ref, out_ref):
    w = w_ref[...]
    acc = (b0_ref[...] * w[0:1, :] + b1_ref[...] * w[1:2, :]
           + b2_ref[...] * w[2:3, :] + bias_ref[...])
    v = _silu(acc)
    out_ref[...] = jnp.concatenate([v, jnp.zeros_like(v)], axis=1)


def _bond_proj(bond_features, w_bp, b_bp, ch=4000):
    e = bond_features.shape[0]
    ch = ch if e % ch == 0 else e
    u = w_bp.shape[1]
    cols = [bond_features[:, k].reshape(e, 1) for k in range(3)]
    return pl.pallas_call(
        _bondproj_body,
        grid=(e // ch,),
        in_specs=[
            pl.BlockSpec((ch, 1), lambda i: (i, 0)),
            pl.BlockSpec((ch, 1), lambda i: (i, 0)),
            pl.BlockSpec((ch, 1), lambda i: (i, 0)),
            pl.BlockSpec((3, u), lambda i: (0, 0)),
            pl.BlockSpec((1, u), lambda i: (0, 0)),
        ],
        out_specs=pl.BlockSpec((ch, 2 * u), lambda i: (i, 0)),
        out_shape=jax.ShapeDtypeStruct((e, 2 * u), jnp.float32),
    )(*cols, w_bp, b_bp.reshape(1, u))


def _gate_body(len_ref, feat_ref, et_ref, wup_ref, bup_ref, out_ref):
    ln = len_ref[...]
    x = feat_ref[...]
    inv = 1.0 / (ln + 1e-8)
    sb = [jnp.sin((float(n) * jnp.pi / CUT3) * ln) * inv
          for n in range(1, MAXN + 1)]
    pows = [jnp.ones_like(x)]
    for _ in range(NSHF - 1):
        pows.append(pows[-1] * x)
    shf = jnp.concatenate(pows, axis=1)
    basis = jnp.concatenate([sb[0] * shf, sb[1] * shf, sb[2] * shf], axis=1)
    et = et_ref[...][:, :wup_ref.shape[0]]
    w = _sigmoid(jnp.dot(et, wup_ref[...],
                         preferred_element_type=jnp.float32) + bup_ref[...])
    out_ref[...] = basis * w


def _gate(lengths, feats, e_t, w_up, b_up, ch=2000):
    t = lengths.shape[0]
    ch = ch if t % ch == 0 else t
    u = e_t.shape[1] // 2
    return pl.pallas_call(
        _gate_body,
        grid=(t // ch,),
        in_specs=[
            pl.BlockSpec((ch, 1), lambda i: (i, 0)),
            pl.BlockSpec((ch, 1), lambda i: (i, 0)),
            pl.BlockSpec((ch, 2 * u), lambda i: (i, 0)),
            pl.BlockSpec((u, NRBF), lambda i: (0, 0)),
            pl.BlockSpec((1, NRBF), lambda i: (0, 0)),
        ],
        out_specs=pl.BlockSpec((ch, NRBF), lambda i: (i, 0)),
        out_shape=jax.ShapeDtypeStruct((t, NRBF), jnp.float32),
    )(lengths.reshape(t, 1), feats.reshape(t, 1), e_t, w_up,
      b_up.reshape(1, NRBF))


def _eupdate_body(e_ref, agg_ref, gs_ref, gd_ref, w3g_ref, b3g_ref, w3v_ref,
                  b3v_ref, w3_ref, bca_ref, out_ref):
    u = w3_ref.shape[0]
    a = agg_ref[...]
    e2 = e_ref[...][:, :u] + _sigmoid(
        jnp.dot(a, w3g_ref[...], preferred_element_type=jnp.float32)
        + b3g_ref[...]) * _silu(
        jnp.dot(a, w3v_ref[...], preferred_element_type=jnp.float32)
        + b3v_ref[...])
    cat = (gs_ref[...][:, :u] + gd_ref[...][:, :u]
           + jnp.dot(e2, w3_ref[...], preferred_element_type=jnp.float32)
           + bca_ref[...])
    v = e2 + _silu(cat)
    out_ref[...] = jnp.concatenate([v, jnp.zeros_like(v)], axis=1)


def _e_update(e, agg, gs, gd, w3g, b3g, w3v, b3v, w3, bca, ch=4000):
    n = e.shape[0]
    ch = ch if n % ch == 0 else n
    u = e.shape[1] // 2
    return pl.pallas_call(
        _eupdate_body,
        grid=(n // ch,),
        in_specs=[
            pl.BlockSpec((ch, 2 * u), lambda i: (i, 0)),
            pl.BlockSpec((ch, NRBF), lambda i: (i, 0)),
            pl.BlockSpec((ch, 2 * u), lambda i: (i, 0)),
            pl.BlockSpec((ch, 2 * u), lambda i: (i, 0)),
            pl.BlockSpec((NRBF, u), lambda i: (0, 0)),
            pl.BlockSpec((1, u), lambda i: (0, 0)),
            pl.BlockSpec((NRBF, u), lambda i: (0, 0)),
            pl.BlockSpec((1, u), lambda i: (0, 0)),
            pl.BlockSpec((u, u), lambda i: (0, 0)),
            pl.BlockSpec((1, u), lambda i: (0, 0)),
        ],
        out_specs=pl.BlockSpec((ch, 2 * u), lambda i: (i, 0)),
        out_shape=jax.ShapeDtypeStruct((n, 2 * u), jnp.float32),
    )(e, agg, gs, gd, w3g, b3g.reshape(1, u), w3v, b3v.reshape(1, u), w3,
      bca.reshape(1, u))


def _proj2_body(x_ref, w1_ref, w2_ref, o1_ref, o2_ref):
    x = x_ref[...]
    o1 = jnp.dot(x, w1_ref[...], preferred_element_type=jnp.float32)
    o2 = jnp.dot(x, w2_ref[...], preferred_element_type=jnp.float32)
    o1_ref[...] = jnp.concatenate([o1, jnp.zeros_like(o1)], axis=1)
    o2_ref[...] = jnp.concatenate([o2, jnp.zeros_like(o2)], axis=1)


def _proj2(x, w1, w2, ch=2000):
    n, u = x.shape
    ch = ch if n % ch == 0 else n
    return pl.pallas_call(
        _proj2_body,
        grid=(n // ch,),
        in_specs=[
            pl.BlockSpec((ch, u), lambda i: (i, 0)),
            pl.BlockSpec((u, u), lambda i: (0, 0)),
            pl.BlockSpec((u, u), lambda i: (0, 0)),
        ],
        out_specs=[
            pl.BlockSpec((ch, 2 * u), lambda i: (i, 0)),
            pl.BlockSpec((ch, 2 * u), lambda i: (i, 0)),
        ],
        out_shape=[
            jax.ShapeDtypeStruct((n, 2 * u), jnp.float32),
            jax.ShapeDtypeStruct((n, 2 * u), jnp.float32),
        ],
    )(x, w1, w2)


def _xupdate_body(x_ref, m0_ref, m1_ref, wag_ref, bag_ref, wav_ref, bav_ref,
                  out_ref):
    u = wag_ref.shape[0]
    m = m0_ref[...][:, :u] + m1_ref[...][:, :u]
    out_ref[...] = x_ref[...] + _sigmoid(
        jnp.dot(m, wag_ref[...], preferred_element_type=jnp.float32)
        + bag_ref[...]) * _silu(
        jnp.dot(m, wav_ref[...], preferred_element_type=jnp.float32)
        + bav_ref[...])


def _x_update(x, m0, m1, wag, bag, wav, bav, ch=2000):
    n, u = x.shape
    ch = ch if n % ch == 0 else n
    nb = n // ch
    return pl.pallas_call(
        _xupdate_body,
        grid=(nb,),
        in_specs=[
            pl.BlockSpec((ch, u), lambda i: (i, 0)),
            pl.BlockSpec((ch, 2 * u), lambda i: (i, 0)),
            pl.BlockSpec((ch, 2 * u), lambda i: (i, 0)),
            pl.BlockSpec((u, u), lambda i: (0, 0)),
            pl.BlockSpec((1, u), lambda i: (0, 0)),
            pl.BlockSpec((u, u), lambda i: (0, 0)),
            pl.BlockSpec((1, u), lambda i: (0, 0)),
        ],
        out_specs=pl.BlockSpec((ch, u), lambda i: (i, 0)),
        out_shape=jax.ShapeDtypeStruct((n, u), jnp.float32),
    )(x, m0, m1, wag, bag.reshape(1, u), wav, bav.reshape(1, u))


def _readout_body(x_ref, wro_ref, bro_ref, wf1_ref, bf1_ref, wf2_ref,
                  bf2_ref, out_ref, *, groups):
    x = x_ref[...]
    n, u = x.shape
    w = _sigmoid(jnp.dot(x, wro_ref[...], preferred_element_type=jnp.float32)
                 + bro_ref[...])
    p = (w * x).reshape(groups, n // groups, u)
    r = jnp.sum(p, axis=1)
    h = _silu(jnp.dot(r, wf1_ref[...], preferred_element_type=jnp.float32)
              + bf1_ref[...])
    o = jnp.dot(h, wf2_ref[...], preferred_element_type=jnp.float32) \
        + bf2_ref[...]
    out_ref[...] = o * OUT_STD + OUT_MEAN


def _readout(x, groups, wro, bro, wf1, bf1, wf2, bf2):
    n, u = x.shape
    body = functools.partial(_readout_body, groups=groups)
    return pl.pallas_call(
        body,
        in_specs=[
            pl.BlockSpec((n, u), lambda: (0, 0)),
            pl.BlockSpec((u, u), lambda: (0, 0)),
            pl.BlockSpec((1, u), lambda: (0, 0)),
            pl.BlockSpec((u, u), lambda: (0, 0)),
            pl.BlockSpec((1, u), lambda: (0, 0)),
            pl.BlockSpec((u, 1), lambda: (0, 0)),
            pl.BlockSpec((1, 1), lambda: (0, 0)),
        ],
        out_specs=pl.BlockSpec((groups, 1), lambda: (0, 0)),
        out_shape=jax.ShapeDtypeStruct((groups, 1), jnp.float32),
    )(x, wro, bro.reshape(1, u), wf1, bf1.reshape(1, u), wf2,
      bf2.reshape(1, 1))


def _sc_mesh():
    return plsc.VectorSubcoreMesh(core_axis_name="c", subcore_axis_name="s")


def _gather_rows(table, idx):
    b = idx.shape[0]
    u = table.shape[1]
    nblk = b // _BLK
    assert b % _BLK == 0

    @functools.partial(
        pl.kernel,
        mesh=_sc_mesh(),
        out_type=jax.ShapeDtypeStruct((b, u), jnp.float32),
        scratch_types=[
            pltpu.VMEM((_BLK,), jnp.int32),
            pltpu.VMEM((_BLK, u), jnp.float32),
            pltpu.SemaphoreType.DMA,
        ],
    )
    def k(table_hbm, idx_hbm, out_hbm, idx_v, rows_v, sem):
        wid = lax.axis_index("s") * _NC + lax.axis_index("c")
        cnt = (nblk - wid + _NW - 1) // _NW

        def body(i, carry):
            base = (wid + i * _NW) * _BLK
            pltpu.sync_copy(idx_hbm.at[pl.ds(base, _BLK)], idx_v)
            pltpu.async_copy(table_hbm.at[idx_v], rows_v, sem).wait()
            pltpu.sync_copy(rows_v, out_hbm.at[pl.ds(base, _BLK)])
            return carry

        lax.fori_loop(0, cnt, body, 0)

    return k(table, idx)


def _sc_scatter_atoms(vals, idx, n_out):
    e, u = vals.shape
    nblk = e // _BLK
    assert e % _BLK == 0
    np_ = ((n_out + _BLK - 1) // _BLK) * _BLK
    ablk = np_ // _BLK

    @functools.partial(
        pl.kernel,
        mesh=_sc_mesh(),
        out_type=jax.ShapeDtypeStruct((2 * np_, u), jnp.float32),
        scratch_types=[
            pltpu.VMEM((_BLK,), jnp.int32),
            pltpu.VMEM((_BLK, u), jnp.float32),
            pltpu.VMEM_SHARED((np_, u), jnp.float32),
            pltpu.SemaphoreType.DMA,
        ],
    )
    def k(vals_hbm, idx_hbm, out_hbm, idx_v, rows_v, acc, sem):
        c = lax.axis_index("c")
        s = lax.axis_index("s")
        wid = s * _NC + c

        def zrow(r, carry):
            for g in range(u // 16):
                rows_v[r, pl.ds(g * 16, 16)] = jnp.zeros((16,), jnp.float32)
            return carry

        lax.fori_loop(0, _BLK, zrow, 0)
        zcnt = (ablk - s + _NS - 1) // _NS

        def zblk(i, carry):
            pltpu.sync_copy(rows_v, acc.at[pl.ds((s + i * _NS) * _BLK, _BLK)])
            return carry

        lax.fori_loop(0, zcnt, zblk, 0)
        plsc.subcore_barrier()

        cnt = (nblk - wid + _NW - 1) // _NW

        def body(i, carry):
            base = (wid + i * _NW) * _BLK
            pltpu.sync_copy(idx_hbm.at[pl.ds(base, _BLK)], idx_v)
            pltpu.sync_copy(vals_hbm.at[pl.ds(base, _BLK)], rows_v)
            pltpu.sync_copy(rows_v, acc.at[idx_v], add=True)
            return carry

        lax.fori_loop(0, cnt, body, 0)
        plsc.subcore_barrier()

        def dblk(i, carry):
            off = (s + i * _NS) * _BLK
            pltpu.sync_copy(acc.at[pl.ds(off, _BLK)],
                            out_hbm.at[pl.ds(c * np_ + off, _BLK)])
            return carry

        lax.fori_loop(0, zcnt, dblk, 0)

    return k(vals, idx)


def _segsum(vals, idx, num):
    return jax.ops.segment_sum(vals, idx, num_segments=num)


_BSHIFT = 15
_BSPAN = 1 << _BSHIFT
_NBUCK = 10
_ACCROWS = _BSPAN + _BLK
_STG = 144


def _lane_val(vec, k):
    sel = (lax.iota(jnp.int32, 16) == k).astype(vec.dtype)
    return jnp.sum(vec * sel)


def _sc_hist(ti, t_rows):
    nblk = t_rows // _BLK

    @functools.partial(
        pl.kernel,
        mesh=_sc_mesh(),
        out_type=jax.ShapeDtypeStruct((_NW, 16), jnp.int32),
        scratch_types=[
            pltpu.VMEM((_BLK,), jnp.int32),
            pltpu.VMEM((16,), jnp.int32),
            pltpu.SemaphoreType.DMA,
        ],
    )
    def k(ti_hbm, out_hbm, ti_v, h_v, sem):
        wid = lax.axis_index("s") * _NC + lax.axis_index("c")
        cnt = (nblk - wid + _NW - 1) // _NW

        def body(i, hist):
            base = (wid + i * _NW) * _BLK
            pltpu.sync_copy(ti_hbm.at[pl.ds(base, _BLK)], ti_v)
            for v in range(8):
                bid = lax.shift_right_logical(ti_v[pl.ds(v * 16, 16)],
                                              _BSHIFT)
                for b in range(_NBUCK):
                    m = bid == b
                    c = plsc.all_reduce_population_count(m)
                    oh = (lax.iota(jnp.int32, 16) == b).astype(jnp.int32)
                    hist = hist + oh * c
            return hist

        hist = lax.fori_loop(0, cnt, body, jnp.zeros((16,), jnp.int32))
        h_v[...] = hist
        pltpu.sync_copy(h_v, out_hbm.at[wid])

    return k(ti)


def _offsets_body(h_ref, bases_ref, bounds_ref, *, tp):
    h = h_ref[...]
    pb = (h + _BLK - 1) // _BLK
    pbf = pb.astype(jnp.float32)
    nw, nb = h.shape
    colsum = jnp.sum(pbf, axis=0, keepdims=True)
    r = jax.lax.broadcasted_iota(jnp.int32, (nb, nb), 0)
    c = jax.lax.broadcasted_iota(jnp.int32, (nb, nb), 1)
    lmat = (r < c).astype(jnp.float32)
    bstart = jnp.dot(colsum, lmat,
                     preferred_element_type=jnp.float32)
    rw = jax.lax.broadcasted_iota(jnp.int32, (nw, nw), 0)
    cw = jax.lax.broadcasted_iota(jnp.int32, (nw, nw), 1)
    wmat = (cw < rw).astype(jnp.float32)
    wpre = jnp.dot(wmat, pbf, preferred_element_type=jnp.float32)
    bases = ((bstart + wpre) * float(_BLK)).astype(jnp.int32)
    bases_ref[...] = bases
    bend = bstart + colsum
    filled = jnp.sum(colsum, keepdims=True)
    fillb = jnp.broadcast_to(filled, (1, nb))
    tailb = float(tp // _BLK) - fillb
    z = jnp.zeros((4, nb), jnp.float32)
    bounds = jnp.concatenate([bstart, bend, fillb, tailb, z], axis=0)
    bounds_ref[...] = bounds.astype(jnp.int32)


def _tc_offsets(hist, tp):
    nw, nb = hist.shape
    body = functools.partial(_offsets_body, tp=tp)
    return pl.pallas_call(
        body,
        in_specs=[pl.BlockSpec((nw, nb), lambda: (0, 0))],
        out_specs=[
            pl.BlockSpec((nw, nb), lambda: (0, 0)),
            pl.BlockSpec((8, nb), lambda: (0, 0)),
        ],
        out_shape=[
            jax.ShapeDtypeStruct((nw, nb), jnp.int32),
            jax.ShapeDtypeStruct((8, nb), jnp.int32),
        ],
    )(hist)


def _sc_permute(ti, tj, lens, feats, bases, bounds_flat, tp):
    t_rows = ti.shape[0]
    nblk = t_rows // _BLK

    @functools.partial(
        pl.kernel,
        mesh=_sc_mesh(),
        out_type=[
            jax.ShapeDtypeStruct((tp,), jnp.int32),
            jax.ShapeDtypeStruct((tp,), jnp.int32),
            jax.ShapeDtypeStruct((tp,), jnp.float32),
            jax.ShapeDtypeStruct((tp,), jnp.float32),
        ],
        scratch_types=[
            pltpu.VMEM((_BLK,), jnp.int32),
            pltpu.VMEM((_BLK,), jnp.int32),
            pltpu.VMEM((_BLK,), jnp.float32),
            pltpu.VMEM((_BLK,), jnp.float32),
            pltpu.VMEM((2 * _NBUCK * _STG,), jnp.int32),
            pltpu.VMEM((2 * _NBUCK * _STG,), jnp.float32),
            pltpu.VMEM((16,), jnp.int32),
            pltpu.VMEM((128,), jnp.int32),
            pltpu.VMEM((_BLK,), jnp.int32),
            pltpu.VMEM((_BLK,), jnp.float32),
            pltpu.SemaphoreType.DMA,
        ],
    )
    def k(ti_hbm, tj_hbm, ln_hbm, ft_hbm, bs_hbm, bd_hbm,
          tio_hbm, tjo_hbm, lno_hbm, fto_hbm,
          ti_v, tj_v, ln_v, ft_v, stg_i, stg_f, bas_v, bnd_v,
          dmp_i, dmp_f, sem):
        wid = lax.axis_index("s") * _NC + lax.axis_index("c")
        pltpu.sync_copy(bs_hbm.at[wid], bas_v)
        pltpu.sync_copy(bd_hbm, bnd_v)
        bvec = bas_v[...]
        iota = lax.iota(jnp.int32, 16)

        for g in range(8):
            dmp_i[pl.ds(g * 16, 16)] = (iota + g * 16) * 977
            dmp_f[pl.ds(g * 16, 16)] = jnp.ones((16,), jnp.float32)

        cnt = (nblk - wid + _NW - 1) // _NW
        init = [jnp.zeros((), jnp.int32)] * (2 * _NBUCK)
        for b in range(_NBUCK):
            init[_NBUCK + b] = _lane_val(bvec, b)

        def body(i, carry):
            carry = list(carry)
            base = (wid + i * _NW) * _BLK
            pltpu.sync_copy(ti_hbm.at[pl.ds(base, _BLK)], ti_v)
            pltpu.sync_copy(tj_hbm.at[pl.ds(base, _BLK)], tj_v)
            pltpu.sync_copy(ln_hbm.at[pl.ds(base, _BLK)], ln_v)
            pltpu.sync_copy(ft_hbm.at[pl.ds(base, _BLK)], ft_v)
            for v in range(8):
                tiv = ti_v[pl.ds(v * 16, 16)]
                tjv = tj_v[pl.ds(v * 16, 16)]
                lnv = ln_v[pl.ds(v * 16, 16)]
                ftv = ft_v[pl.ds(v * 16, 16)]
                bid = tiv // _BSPAN
                for b in range(_NBUCK):
                    fill = carry[b]
                    cur = carry[_NBUCK + b]
                    m = bid == b
                    cum = plsc.cumsum(m.astype(jnp.int32))
                    pos = fill + cum - 1
                    plsc.store_scatter(stg_i, [pos + b * _STG], tiv, mask=m)
                    plsc.store_scatter(stg_i,
                                       [pos + (_NBUCK + b) * _STG], tjv,
                                       mask=m)
                    plsc.store_scatter(stg_f, [pos + b * _STG], lnv, mask=m)
                    plsc.store_scatter(stg_f,
                                       [pos + (_NBUCK + b) * _STG], ftv,
                                       mask=m)
                    fill = fill + jnp.max(cum)
                    dof = fill >= _BLK

                    @pl.when(dof)
                    def _():
                        pltpu.sync_copy(stg_i.at[pl.ds(b * _STG, _BLK)],
                                        tio_hbm.at[pl.ds(cur, _BLK)])
                        pltpu.sync_copy(
                            stg_i.at[pl.ds((_NBUCK + b) * _STG, _BLK)],
                            tjo_hbm.at[pl.ds(cur, _BLK)])
                        pltpu.sync_copy(stg_f.at[pl.ds(b * _STG, _BLK)],
                                        lno_hbm.at[pl.ds(cur, _BLK)])
                        pltpu.sync_copy(
                            stg_f.at[pl.ds((_NBUCK + b) * _STG, _BLK)],
                            fto_hbm.at[pl.ds(cur, _BLK)])
                        stg_i[pl.ds(b * _STG, 16)] = \
                            stg_i[pl.ds(b * _STG + _BLK, 16)]
                        stg_i[pl.ds((_NBUCK + b) * _STG, 16)] = \
                            stg_i[pl.ds((_NBUCK + b) * _STG + _BLK, 16)]
                        stg_f[pl.ds(b * _STG, 16)] = \
                            stg_f[pl.ds(b * _STG + _BLK, 16)]
                        stg_f[pl.ds((_NBUCK + b) * _STG, 16)] = \
                            stg_f[pl.ds((_NBUCK + b) * _STG + _BLK, 16)]

                    carry[b] = jnp.where(dof, fill - _BLK, fill)
                    carry[_NBUCK + b] = jnp.where(dof, cur + _BLK, cur)
            return tuple(carry)

        fin = lax.fori_loop(0, cnt, body, tuple(init))

        for b in range(_NBUCK):
            fill = fin[b]
            cur = fin[_NBUCK + b]
            dump_ti = (b * _BSPAN + _BSPAN) + iota
            dump_tj = iota * 977
            ones = jnp.ones((16,), jnp.float32)
            for g in range(8):
                pos = fill + g * 16 + iota
                m = pos < _BLK
                plsc.store_scatter(stg_i, [pos + b * _STG], dump_ti, mask=m)
                plsc.store_scatter(stg_i, [pos + (_NBUCK + b) * _STG],
                                   dump_tj, mask=m)
                plsc.store_scatter(stg_f, [pos + b * _STG], ones, mask=m)
                plsc.store_scatter(stg_f, [pos + (_NBUCK + b) * _STG],
                                   ones, mask=m)

            @pl.when(fill > 0)
            def _():
                pltpu.sync_copy(stg_i.at[pl.ds(b * _STG, _BLK)],
                                tio_hbm.at[pl.ds(cur, _BLK)])
                pltpu.sync_copy(stg_i.at[pl.ds((_NBUCK + b) * _STG, _BLK)],
                                tjo_hbm.at[pl.ds(cur, _BLK)])
                pltpu.sync_copy(stg_f.at[pl.ds(b * _STG, _BLK)],
                                lno_hbm.at[pl.ds(cur, _BLK)])
                pltpu.sync_copy(stg_f.at[pl.ds((_NBUCK + b) * _STG, _BLK)],
                                fto_hbm.at[pl.ds(cur, _BLK)])

        fblk = _lane_val(bnd_v[pl.ds(32, 16)], 0)
        tblk = _lane_val(bnd_v[pl.ds(48, 16)], 0)
        tcnt = (tblk - wid + _NW - 1) // _NW

        def tbody(i, carry):
            off = (fblk + wid + i * _NW) * _BLK
            pltpu.sync_copy(dmp_i, tio_hbm.at[pl.ds(off, _BLK)])
            pltpu.sync_copy(dmp_i, tjo_hbm.at[pl.ds(off, _BLK)])
            pltpu.sync_copy(dmp_f, lno_hbm.at[pl.ds(off, _BLK)])
            pltpu.sync_copy(dmp_f, fto_hbm.at[pl.ds(off, _BLK)])
            return carry

        lax.fori_loop(0, tcnt, tbody, 0)

    return k(ti, tj, lens, feats, bases, bounds_flat)


def _sc_scatter_pair(msg, ti_p, bounds, k, e_rows):
    w = msg.shape[1]
    ablk = _ACCROWS // _BLK
    sz0 = min(_BSPAN, e_rows - 2 * k * _BSPAN) // _NS
    sz1 = min(_BSPAN, e_rows - (2 * k + 1) * _BSPAN) // _NS

    @functools.partial(
        pl.kernel,
        mesh=_sc_mesh(),
        out_type=jax.ShapeDtypeStruct((2 * _BSPAN, w), jnp.float32),
        scratch_types=[
            pltpu.VMEM((_BLK,), jnp.int32),
            pltpu.VMEM((_BLK,), jnp.int32),
            pltpu.VMEM((_BLK, w), jnp.float32),
            pltpu.VMEM((128,), jnp.int32),
            pltpu.VMEM_SHARED((_ACCROWS, w), jnp.float32),
            pltpu.SemaphoreType.DMA,
        ],
    )
    def kk(msg_hbm, tip_hbm, bd_hbm, out_hbm, tiv, tloc, rows_v,
           bnd_v, acc, sem):
        c = lax.axis_index("c")
        s = lax.axis_index("s")
        pltpu.sync_copy(bd_hbm, bnd_v)
        bs0 = _lane_val(bnd_v[pl.ds(0, 16)], 2 * k)
        bs1 = _lane_val(bnd_v[pl.ds(0, 16)], 2 * k + 1)
        be0 = _lane_val(bnd_v[pl.ds(16, 16)], 2 * k)
        be1 = _lane_val(bnd_v[pl.ds(16, 16)], 2 * k + 1)
        bs = jnp.where(c == 0, bs0, bs1)
        be = jnp.where(c == 0, be0, be1)

        def zrow(r, carry):
            for g in range(w // 16):
                rows_v[r, pl.ds(g * 16, 16)] = jnp.zeros((16,), jnp.float32)
            return carry

        lax.fori_loop(0, _BLK, zrow, 0)
        zcnt = (ablk - s + _NS - 1) // _NS

        def zblk(i, carry):
            pltpu.sync_copy(rows_v,
                            acc.at[pl.ds((s + i * _NS) * _BLK, _BLK)])
            return carry

        lax.fori_loop(0, zcnt, zblk, 0)
        plsc.subcore_barrier()

        bcnt = jnp.maximum(0, (be - bs - s + _NS - 1) // _NS)

        def body(i, carry):
            base = (bs + s + i * _NS) * _BLK
            pltpu.sync_copy(tip_hbm.at[pl.ds(base, _BLK)], tiv)
            for g in range(8):
                tloc[pl.ds(g * 16, 16)] = (
                    tiv[pl.ds(g * 16, 16)] - (2 * k + c) * _BSPAN)
            pltpu.sync_copy(msg_hbm.at[pl.ds(base, _BLK)], rows_v)
            pltpu.sync_copy(rows_v, acc.at[tloc], add=True)
            return carry

        lax.fori_loop(0, bcnt, body, 0)
        plsc.subcore_barrier()

        @pl.when(c == 0)
        def _():
            pltpu.sync_copy(acc.at[pl.ds(s * sz0, sz0)],
                            out_hbm.at[pl.ds(s * sz0, sz0)])

        @pl.when(c == 1)
        def _():
            pltpu.sync_copy(acc.at[pl.ds(s * sz1, sz1)],
                            out_hbm.at[pl.ds(_BSPAN + s * sz1, sz1)])

    return kk(msg, ti_p, bounds)


def _sc_scatter_big(msg, ti_p, bounds, e_rows):
    parts = []
    for k in range(_NBUCK // 2):
        pk = _sc_scatter_pair(msg, ti_p, bounds, k, e_rows)
        take = min(2 * _BSPAN, e_rows - 2 * k * _BSPAN)
        parts.append(pk if take == 2 * _BSPAN else pk[:take])
    return jnp.concatenate(parts, axis=0)


def kernel(atom_types, edge_index, bond_features, triple_bond_indices,
           triple_bond_lengths, triple_features, n_atoms, atom_embedding,
           W_bp, b_bp, W_up, b_up, W_3g, b_3g, W_3v, b_3v, W_ca, b_ca,
           W_ag, b_ag, W_av, b_av, W_ro, b_ro, W_f1, b_f1, W_f2, b_f2):
    n = atom_types.shape[0]
    e_cnt = edge_index.shape[1]
    u = atom_embedding.shape[1]
    g_cnt = n_atoms.shape[0]
    nblocks = W_up.shape[0]

    src = edge_index[0]
    dst = edge_index[1]
    ti = triple_bond_indices[:, 0]
    tj = triple_bond_indices[:, 1]

    x = _embed(atom_types, atom_embedding)
    e = _bond_proj(bond_features, W_bp, b_bp)

    for b in range(nblocks):
        e_t = _gather_rows(e, tj)
        msg = _gate(triple_bond_lengths, triple_features, e_t, W_up[b],
                    b_up[b], ch=2000)
        agg = _segsum(msg, ti, e_cnt)
        w1 = W_ca[b][:u]
        w2 = W_ca[b][u:2 * u]
        w3 = W_ca[b][2 * u:]
        xs, xd = _proj2(x, w1, w2)
        gs = _gather_rows(xs, src)
        gd = _gather_rows(xd, dst)
        e = _e_update(e, agg, gs, gd, W_3g[b], b_3g[b], W_3v[b], b_3v[b], w3,
                      b_ca[b])
        m = _segsum(e, dst, n)
        x = _x_update(x, m, jnp.zeros_like(m), W_ag[b], b_ag[b],
                      W_av[b], b_av[b])

    return _readout(x, g_cnt, W_ro, b_ro, W_f1, b_f1, W_f2, b_f2)


# speedup vs baseline: 1.0219x; 1.0219x over previous
import functools

import jax
import jax.numpy as jnp
from jax import lax
from jax.experimental import pallas as pl
from jax.experimental.pallas import tpu as pltpu
from jax.experimental.pallas import tpu_sc as plsc

_NC = 2
_NS = 16
_NW = _NC * _NS
_BLK = 128

MAXN = 3
NSHF = 16
NRBF = 48
CUT3 = 4.0
OUT_STD = 1.0
OUT_MEAN = 0.0


def _sigmoid(v):
    return jax.nn.sigmoid(v)


def _silu(v):
    return v * jax.nn.sigmoid(v)


def _embed_body(types_ref, table_ref, out_ref):
    t = types_ref[...]
    ntypes = table_ref.shape[0]
    iota = jax.lax.broadcasted_iota(jnp.int32, (t.shape[0], ntypes), 1)
    onehot = (t == iota).astype(jnp.float32)
    out_ref[...] = jnp.dot(onehot, table_ref[...],
                           preferred_element_type=jnp.float32)


def _embed(atom_types, table, ch=2000):
    n = atom_types.shape[0]
    ch = ch if n % ch == 0 else n
    u = table.shape[1]
    t2 = atom_types.reshape(n, 1).astype(jnp.int32)
    return pl.pallas_call(
        _embed_body,
        grid=(n // ch,),
        in_specs=[
            pl.BlockSpec((ch, 1), lambda i: (i, 0)),
            pl.BlockSpec(table.shape, lambda i: (0, 0)),
        ],
        out_specs=pl.BlockSpec((ch, u), lambda i: (i, 0)),
        out_shape=jax.ShapeDtypeStruct((n, u), jnp.float32),
    )(t2, table)


def _bondproj_body(b0_ref, b1_ref, b2_ref, w_ref, bias_ref, out_ref):
    w = w_ref[...]
    acc = (b0_ref[...] * w[0:1, :] + b1_ref[...] * w[1:2, :]
           + b2_ref[...] * w[2:3, :] + bias_ref[...])
    v = _silu(acc)
    out_ref[...] = jnp.concatenate([v, jnp.zeros_like(v)], axis=1)


def _bond_proj(bond_features, w_bp, b_bp, ch=4000):
    e = bond_features.shape[0]
    ch = ch if e % ch == 0 else e
    u = w_bp.shape[1]
    cols = [bond_features[:, k].reshape(e, 1) for k in range(3)]
    return pl.pallas_call(
        _bondproj_body,
        grid=(e // ch,),
        in_specs=[
            pl.BlockSpec((ch, 1), lambda i: (i, 0)),
            pl.BlockSpec((ch, 1), lambda i: (i, 0)),
            pl.BlockSpec((ch, 1), lambda i: (i, 0)),
            pl.BlockSpec((3, u), lambda i: (0, 0)),
            pl.BlockSpec((1, u), lambda i: (0, 0)),
        ],
        out_specs=pl.BlockSpec((ch, 2 * u), lambda i: (i, 0)),
        out_shape=jax.ShapeDtypeStruct((e, 2 * u), jnp.float32),
    )(*cols, w_bp, b_bp.reshape(1, u))


def _gate_body(len_ref, feat_ref, et_ref, wup_ref, bup_ref, out_ref):
    ln = len_ref[...]
    x = feat_ref[...]
    inv = 1.0 / (ln + 1e-8)
    sb = [jnp.sin((float(n) * jnp.pi / CUT3) * ln) * inv
          for n in range(1, MAXN + 1)]
    pows = [jnp.ones_like(x)]
    for _ in range(NSHF - 1):
        pows.append(pows[-1] * x)
    shf = jnp.concatenate(pows, axis=1)
    basis = jnp.concatenate([sb[0] * shf, sb[1] * shf, sb[2] * shf], axis=1)
    et = et_ref[...][:, :wup_ref.shape[0]]
    w = _sigmoid(jnp.dot(et, wup_ref[...],
                         preferred_element_type=jnp.float32) + bup_ref[...])
    out_ref[...] = basis * w


def _gate(lengths, feats, e_t, w_up, b_up, ch=2000):
    t = lengths.shape[0]
    ch = ch if t % ch == 0 else t
    u = e_t.shape[1] // 2
    return pl.pallas_call(
        _gate_body,
        grid=(t // ch,),
        in_specs=[
            pl.BlockSpec((ch, 1), lambda i: (i, 0)),
            pl.BlockSpec((ch, 1), lambda i: (i, 0)),
            pl.BlockSpec((ch, 2 * u), lambda i: (i, 0)),
            pl.BlockSpec((u, NRBF), lambda i: (0, 0)),
            pl.BlockSpec((1, NRBF), lambda i: (0, 0)),
        ],
        out_specs=pl.BlockSpec((ch, NRBF), lambda i: (i, 0)),
        out_shape=jax.ShapeDtypeStruct((t, NRBF), jnp.float32),
    )(lengths.reshape(t, 1), feats.reshape(t, 1), e_t, w_up,
      b_up.reshape(1, NRBF))


def _eupdate_body(e_ref, agg_ref, gs_ref, gd_ref, w3g_ref, b3g_ref, w3v_ref,
                  b3v_ref, w3_ref, bca_ref, out_ref):
    u = w3_ref.shape[0]
    a = agg_ref[...]
    e2 = e_ref[...][:, :u] + _sigmoid(
        jnp.dot(a, w3g_ref[...], preferred_element_type=jnp.float32)
        + b3g_ref[...]) * _silu(
        jnp.dot(a, w3v_ref[...], preferred_element_type=jnp.float32)
        + b3v_ref[...])
    cat = (gs_ref[...][:, :u] + gd_ref[...][:, :u]
           + jnp.dot(e2, w3_ref[...], preferred_element_type=jnp.float32)
           + bca_ref[...])
    v = e2 + _silu(cat)
    out_ref[...] = jnp.concatenate([v, jnp.zeros_like(v)], axis=1)


def _e_update(e, agg, gs, gd, w3g, b3g, w3v, b3v, w3, bca, ch=4000):
    n = e.shape[0]
    ch = ch if n % ch == 0 else n
    u = e.shape[1] // 2
    return pl.pallas_call(
        _eupdate_body,
        grid=(n // ch,),
        in_specs=[
            pl.BlockSpec((ch, 2 * u), lambda i: (i, 0)),
            pl.BlockSpec((ch, NRBF), lambda i: (i, 0)),
            pl.BlockSpec((ch, 2 * u), lambda i: (i, 0)),
            pl.BlockSpec((ch, 2 * u), lambda i: (i, 0)),
            pl.BlockSpec((NRBF, u), lambda i: (0, 0)),
            pl.BlockSpec((1, u), lambda i: (0, 0)),
            pl.BlockSpec((NRBF, u), lambda i: (0, 0)),
            pl.BlockSpec((1, u), lambda i: (0, 0)),
            pl.BlockSpec((u, u), lambda i: (0, 0)),
            pl.BlockSpec((1, u), lambda i: (0, 0)),
        ],
        out_specs=pl.BlockSpec((ch, 2 * u), lambda i: (i, 0)),
        out_shape=jax.ShapeDtypeStruct((n, 2 * u), jnp.float32),
    )(e, agg, gs, gd, w3g, b3g.reshape(1, u), w3v, b3v.reshape(1, u), w3,
      bca.reshape(1, u))


def _proj2_body(x_ref, w1_ref, w2_ref, o1_ref, o2_ref):
    x = x_ref[...]
    o1 = jnp.dot(x, w1_ref[...], preferred_element_type=jnp.float32)
    o2 = jnp.dot(x, w2_ref[...], preferred_element_type=jnp.float32)
    o1_ref[...] = jnp.concatenate([o1, jnp.zeros_like(o1)], axis=1)
    o2_ref[...] = jnp.concatenate([o2, jnp.zeros_like(o2)], axis=1)


def _proj2(x, w1, w2, ch=2000):
    n, u = x.shape
    ch = ch if n % ch == 0 else n
    return pl.pallas_call(
        _proj2_body,
        grid=(n // ch,),
        in_specs=[
            pl.BlockSpec((ch, u), lambda i: (i, 0)),
            pl.BlockSpec((u, u), lambda i: (0, 0)),
            pl.BlockSpec((u, u), lambda i: (0, 0)),
        ],
        out_specs=[
            pl.BlockSpec((ch, 2 * u), lambda i: (i, 0)),
            pl.BlockSpec((ch, 2 * u), lambda i: (i, 0)),
        ],
        out_shape=[
            jax.ShapeDtypeStruct((n, 2 * u), jnp.float32),
            jax.ShapeDtypeStruct((n, 2 * u), jnp.float32),
        ],
    )(x, w1, w2)


def _xupdate_body(x_ref, m0_ref, m1_ref, wag_ref, bag_ref, wav_ref, bav_ref,
                  out_ref):
    u = wag_ref.shape[0]
    m = m0_ref[...][:, :u] + m1_ref[...][:, :u]
    out_ref[...] = x_ref[...] + _sigmoid(
        jnp.dot(m, wag_ref[...], preferred_element_type=jnp.float32)
        + bag_ref[...]) * _silu(
        jnp.dot(m, wav_ref[...], preferred_element_type=jnp.float32)
        + bav_ref[...])


def _x_update(x, m0, m1, wag, bag, wav, bav, ch=2000):
    n, u = x.shape
    ch = ch if n % ch == 0 else n
    nb = n // ch
    return pl.pallas_call(
        _xupdate_body,
        grid=(nb,),
        in_specs=[
            pl.BlockSpec((ch, u), lambda i: (i, 0)),
            pl.BlockSpec((ch, 2 * u), lambda i: (i, 0)),
            pl.BlockSpec((ch, 2 * u), lambda i: (i, 0)),
            pl.BlockSpec((u, u), lambda i: (0, 0)),
            pl.BlockSpec((1, u), lambda i: (0, 0)),
            pl.BlockSpec((u, u), lambda i: (0, 0)),
            pl.BlockSpec((1, u), lambda i: (0, 0)),
        ],
        out_specs=pl.BlockSpec((ch, u), lambda i: (i, 0)),
        out_shape=jax.ShapeDtypeStruct((n, u), jnp.float32),
    )(x, m0, m1, wag, bag.reshape(1, u), wav, bav.reshape(1, u))


def _readout_body(x_ref, wro_ref, bro_ref, wf1_ref, bf1_ref, wf2_ref,
                  bf2_ref, out_ref, *, groups):
    x = x_ref[...]
    n, u = x.shape
    w = _sigmoid(jnp.dot(x, wro_ref[...], preferred_element_type=jnp.float32)
                 + bro_ref[...])
    p = (w * x).reshape(groups, n // groups, u)
    r = jnp.sum(p, axis=1)
    h = _silu(jnp.dot(r, wf1_ref[...], preferred_element_type=jnp.float32)
              + bf1_ref[...])
    o = jnp.dot(h, wf2_ref[...], preferred_element_type=jnp.float32) \
        + bf2_ref[...]
    out_ref[...] = o * OUT_STD + OUT_MEAN


def _readout(x, groups, wro, bro, wf1, bf1, wf2, bf2):
    n, u = x.shape
    body = functools.partial(_readout_body, groups=groups)
    return pl.pallas_call(
        body,
        in_specs=[
            pl.BlockSpec((n, u), lambda: (0, 0)),
            pl.BlockSpec((u, u), lambda: (0, 0)),
            pl.BlockSpec((1, u), lambda: (0, 0)),
            pl.BlockSpec((u, u), lambda: (0, 0)),
            pl.BlockSpec((1, u), lambda: (0, 0)),
            pl.BlockSpec((u, 1), lambda: (0, 0)),
            pl.BlockSpec((1, 1), lambda: (0, 0)),
        ],
        out_specs=pl.BlockSpec((groups, 1), lambda: (0, 0)),
        out_shape=jax.ShapeDtypeStruct((groups, 1), jnp.float32),
    )(x, wro, bro.reshape(1, u), wf1, bf1.reshape(1, u), wf2,
      bf2.reshape(1, 1))


def _sc_mesh():
    return plsc.VectorSubcoreMesh(core_axis_name="c", subcore_axis_name="s")


def _gather_rows(table, idx):
    b = idx.shape[0]
    u = table.shape[1]
    nblk = b // _BLK
    assert b % _BLK == 0

    @functools.partial(
        pl.kernel,
        mesh=_sc_mesh(),
        out_type=jax.ShapeDtypeStruct((b, u), jnp.float32),
        scratch_types=[
            pltpu.VMEM((_BLK,), jnp.int32),
            pltpu.VMEM((_BLK, u), jnp.float32),
            pltpu.SemaphoreType.DMA,
        ],
    )
    def k(table_hbm, idx_hbm, out_hbm, idx_v, rows_v, sem):
        wid = lax.axis_index("s") * _NC + lax.axis_index("c")
        cnt = (nblk - wid + _NW - 1) // _NW

        def body(i, carry):
            base = (wid + i * _NW) * _BLK
            pltpu.sync_copy(idx_hbm.at[pl.ds(base, _BLK)], idx_v)
            pltpu.async_copy(table_hbm.at[idx_v], rows_v, sem).wait()
            pltpu.sync_copy(rows_v, out_hbm.at[pl.ds(base, _BLK)])
            return carry

        lax.fori_loop(0, cnt, body, 0)

    return k(table, idx)


def _sc_scatter_atoms(vals, idx, n_out):
    e, u = vals.shape
    nblk = e // _BLK
    assert e % _BLK == 0
    np_ = ((n_out + _BLK - 1) // _BLK) * _BLK
    ablk = np_ // _BLK

    @functools.partial(
        pl.kernel,
        mesh=_sc_mesh(),
        out_type=jax.ShapeDtypeStruct((2 * np_, u), jnp.float32),
        scratch_types=[
            pltpu.VMEM((_BLK,), jnp.int32),
            pltpu.VMEM((_BLK, u), jnp.float32),
            pltpu.VMEM_SHARED((np_, u), jnp.float32),
            pltpu.SemaphoreType.DMA,
        ],
    )
    def k(vals_hbm, idx_hbm, out_hbm, idx_v, rows_v, acc, sem):
        c = lax.axis_index("c")
        s = lax.axis_index("s")
        wid = s * _NC + c

        def zrow(r, carry):
            for g in range(u // 16):
                rows_v[r, pl.ds(g * 16, 16)] = jnp.zeros((16,), jnp.float32)
            return carry

        lax.fori_loop(0, _BLK, zrow, 0)
        zcnt = (ablk - s + _NS - 1) // _NS

        def zblk(i, carry):
            pltpu.sync_copy(rows_v, acc.at[pl.ds((s + i * _NS) * _BLK, _BLK)])
            return carry

        lax.fori_loop(0, zcnt, zblk, 0)
        plsc.subcore_barrier()

        cnt = (nblk - wid + _NW - 1) // _NW

        def body(i, carry):
            base = (wid + i * _NW) * _BLK
            pltpu.sync_copy(idx_hbm.at[pl.ds(base, _BLK)], idx_v)
            pltpu.sync_copy(vals_hbm.at[pl.ds(base, _BLK)], rows_v)
            pltpu.sync_copy(rows_v, acc.at[idx_v], add=True)
            return carry

        lax.fori_loop(0, cnt, body, 0)
        plsc.subcore_barrier()

        def dblk(i, carry):
            off = (s + i * _NS) * _BLK
            pltpu.sync_copy(acc.at[pl.ds(off, _BLK)],
                            out_hbm.at[pl.ds(c * np_ + off, _BLK)])
            return carry

        lax.fori_loop(0, zcnt, dblk, 0)

    return k(vals, idx)


def _segsum(vals, idx, num):
    return jax.ops.segment_sum(vals, idx, num_segments=num)


_BSHIFT = 15
_BSPAN = 1 << _BSHIFT
_NBUCK = 10
_ACCROWS = _BSPAN + _BLK
_STG = 144


def _lane_val(vec, k):
    sel = (lax.iota(jnp.int32, 16) == k).astype(vec.dtype)
    return jnp.sum(vec * sel)


def _sc_hist(ti, t_rows):
    nblk = t_rows // _BLK

    @functools.partial(
        pl.kernel,
        mesh=_sc_mesh(),
        out_type=jax.ShapeDtypeStruct((_NW, 16), jnp.int32),
        scratch_types=[
            pltpu.VMEM((_BLK,), jnp.int32),
            pltpu.VMEM((16,), jnp.int32),
            pltpu.SemaphoreType.DMA,
        ],
    )
    def k(ti_hbm, out_hbm, ti_v, h_v, sem):
        wid = lax.axis_index("s") * _NC + lax.axis_index("c")
        cnt = (nblk - wid + _NW - 1) // _NW

        def body(i, hist):
            base = (wid + i * _NW) * _BLK
            pltpu.sync_copy(ti_hbm.at[pl.ds(base, _BLK)], ti_v)
            for v in range(8):
                bid = lax.shift_right_logical(ti_v[pl.ds(v * 16, 16)],
                                              _BSHIFT)
                for b in range(_NBUCK):
                    m = bid == b
                    c = plsc.all_reduce_population_count(m)
                    oh = (lax.iota(jnp.int32, 16) == b).astype(jnp.int32)
                    hist = hist + oh * c
            return hist

        hist = lax.fori_loop(0, cnt, body, jnp.zeros((16,), jnp.int32))
        h_v[...] = hist
        pltpu.sync_copy(h_v, out_hbm.at[wid])

    return k(ti)


def _offsets_body(h_ref, bases_ref, bounds_ref, *, tp):
    h = h_ref[...]
    pb = (h + _BLK - 1) // _BLK
    pbf = pb.astype(jnp.float32)
    nw, nb = h.shape
    colsum = jnp.sum(pbf, axis=0, keepdims=True)
    r = jax.lax.broadcasted_iota(jnp.int32, (nb, nb), 0)
    c = jax.lax.broadcasted_iota(jnp.int32, (nb, nb), 1)
    lmat = (r < c).astype(jnp.float32)
    bstart = jnp.dot(colsum, lmat,
                     preferred_element_type=jnp.float32)
    rw = jax.lax.broadcasted_iota(jnp.int32, (nw, nw), 0)
    cw = jax.lax.broadcasted_iota(jnp.int32, (nw, nw), 1)
    wmat = (cw < rw).astype(jnp.float32)
    wpre = jnp.dot(wmat, pbf, preferred_element_type=jnp.float32)
    bases = ((bstart + wpre) * float(_BLK)).astype(jnp.int32)
    bases_ref[...] = bases
    bend = bstart + colsum
    filled = jnp.sum(colsum, keepdims=True)
    fillb = jnp.broadcast_to(filled, (1, nb))
    tailb = float(tp // _BLK) - fillb
    z = jnp.zeros((4, nb), jnp.float32)
    bounds = jnp.concatenate([bstart, bend, fillb, tailb, z], axis=0)
    bounds_ref[...] = bounds.astype(jnp.int32)


def _tc_offsets(hist, tp):
    nw, nb = hist.shape
    body = functools.partial(_offsets_body, tp=tp)
    return pl.pallas_call(
        body,
        in_specs=[pl.BlockSpec((nw, nb), lambda: (0, 0))],
        out_specs=[
            pl.BlockSpec((nw, nb), lambda: (0, 0)),
            pl.BlockSpec((8, nb), lambda: (0, 0)),
        ],
        out_shape=[
            jax.ShapeDtypeStruct((nw, nb), jnp.int32),
            jax.ShapeDtypeStruct((8, nb), jnp.int32),
        ],
    )(hist)


def _sc_permute(ti, tj, lens, feats, bases, bounds_flat, tp):
    t_rows = ti.shape[0]
    nblk = t_rows // _BLK

    @functools.partial(
        pl.kernel,
        mesh=_sc_mesh(),
        out_type=[
            jax.ShapeDtypeStruct((tp,), jnp.int32),
            jax.ShapeDtypeStruct((tp,), jnp.int32),
            jax.ShapeDtypeStruct((tp,), jnp.float32),
            jax.ShapeDtypeStruct((tp,), jnp.float32),
        ],
        scratch_types=[
            pltpu.VMEM((_BLK,), jnp.int32),
            pltpu.VMEM((_BLK,), jnp.int32),
            pltpu.VMEM((_BLK,), jnp.float32),
            pltpu.VMEM((_BLK,), jnp.float32),
            pltpu.VMEM((2 * _NBUCK * _STG,), jnp.int32),
            pltpu.VMEM((2 * _NBUCK * _STG,), jnp.float32),
            pltpu.VMEM((16,), jnp.int32),
            pltpu.VMEM((128,), jnp.int32),
            pltpu.VMEM((_BLK,), jnp.int32),
            pltpu.VMEM((_BLK,), jnp.float32),
            pltpu.SemaphoreType.DMA,
        ],
    )
    def k(ti_hbm, tj_hbm, ln_hbm, ft_hbm, bs_hbm, bd_hbm,
          tio_hbm, tjo_hbm, lno_hbm, fto_hbm,
          ti_v, tj_v, ln_v, ft_v, stg_i, stg_f, bas_v, bnd_v,
          dmp_i, dmp_f, sem):
        wid = lax.axis_index("s") * _NC + lax.axis_index("c")
        pltpu.sync_copy(bs_hbm.at[wid], bas_v)
        pltpu.sync_copy(bd_hbm, bnd_v)
        bvec = bas_v[...]
        iota = lax.iota(jnp.int32, 16)

        for g in range(8):
            dmp_i[pl.ds(g * 16, 16)] = (iota + g * 16) * 977
            dmp_f[pl.ds(g * 16, 16)] = jnp.ones((16,), jnp.float32)

        cnt = (nblk - wid + _NW - 1) // _NW
        init = [jnp.zeros((), jnp.int32)] * (2 * _NBUCK)
        for b in range(_NBUCK):
            init[_NBUCK + b] = _lane_val(bvec, b)

        def body(i, carry):
            carry = list(carry)
            base = (wid + i * _NW) * _BLK
            pltpu.sync_copy(ti_hbm.at[pl.ds(base, _BLK)], ti_v)
            pltpu.sync_copy(tj_hbm.at[pl.ds(base, _BLK)], tj_v)
            pltpu.sync_copy(ln_hbm.at[pl.ds(base, _BLK)], ln_v)
            pltpu.sync_copy(ft_hbm.at[pl.ds(base, _BLK)], ft_v)
            for v in range(8):
                tiv = ti_v[pl.ds(v * 16, 16)]
                tjv = tj_v[pl.ds(v * 16, 16)]
                lnv = ln_v[pl.ds(v * 16, 16)]
                ftv = ft_v[pl.ds(v * 16, 16)]
                bid = tiv // _BSPAN
                for b in range(_NBUCK):
                    fill = carry[b]
                    cur = carry[_NBUCK + b]
                    m = bid == b
                    cum = plsc.cumsum(m.astype(jnp.int32))
                    pos = fill + cum - 1
                    plsc.store_scatter(stg_i, [pos + b * _STG], tiv, mask=m)
                    plsc.store_scatter(stg_i,
                                       [pos + (_NBUCK + b) * _STG], tjv,
                                       mask=m)
                    plsc.store_scatter(stg_f, [pos + b * _STG], lnv, mask=m)
                    plsc.store_scatter(stg_f,
                                       [pos + (_NBUCK + b) * _STG], ftv,
                                       mask=m)
                    fill = fill + jnp.max(cum)
                    dof = fill >= _BLK

                    @pl.when(dof)
                    def _():
                        pltpu.sync_copy(stg_i.at[pl.ds(b * _STG, _BLK)],
                                        tio_hbm.at[pl.ds(cur, _BLK)])
                        pltpu.sync_copy(
                            stg_i.at[pl.ds((_NBUCK + b) * _STG, _BLK)],
                            tjo_hbm.at[pl.ds(cur, _BLK)])
                        pltpu.sync_copy(stg_f.at[pl.ds(b * _STG, _BLK)],
                                        lno_hbm.at[pl.ds(cur, _BLK)])
                        pltpu.sync_copy(
                            stg_f.at[pl.ds((_NBUCK + b) * _STG, _BLK)],
                            fto_hbm.at[pl.ds(cur, _BLK)])
                        stg_i[pl.ds(b * _STG, 16)] = \
                            stg_i[pl.ds(b * _STG + _BLK, 16)]
                        stg_i[pl.ds((_NBUCK + b) * _STG, 16)] = \
                            stg_i[pl.ds((_NBUCK + b) * _STG + _BLK, 16)]
                        stg_f[pl.ds(b * _STG, 16)] = \
                            stg_f[pl.ds(b * _STG + _BLK, 16)]
                        stg_f[pl.ds((_NBUCK + b) * _STG, 16)] = \
                            stg_f[pl.ds((_NBUCK + b) * _STG + _BLK, 16)]

                    carry[b] = jnp.where(dof, fill - _BLK, fill)
                    carry[_NBUCK + b] = jnp.where(dof, cur + _BLK, cur)
            return tuple(carry)

        fin = lax.fori_loop(0, cnt, body, tuple(init))

        for b in range(_NBUCK):
            fill = fin[b]
            cur = fin[_NBUCK + b]
            dump_ti = (b * _BSPAN + _BSPAN) + iota
            dump_tj = iota * 977
            ones = jnp.ones((16,), jnp.float32)
            for g in range(8):
                pos = fill + g * 16 + iota
                m = pos < _BLK
                plsc.store_scatter(stg_i, [pos + b * _STG], dump_ti, mask=m)
                plsc.store_scatter(stg_i, [pos + (_NBUCK + b) * _STG],
                                   dump_tj, mask=m)
                plsc.store_scatter(stg_f, [pos + b * _STG], ones, mask=m)
                plsc.store_scatter(stg_f, [pos + (_NBUCK + b) * _STG],
                                   ones, mask=m)

            @pl.when(fill > 0)
            def _():
                pltpu.sync_copy(stg_i.at[pl.ds(b * _STG, _BLK)],
                                tio_hbm.at[pl.ds(cur, _BLK)])
                pltpu.sync_copy(stg_i.at[pl.ds((_NBUCK + b) * _STG, _BLK)],
                                tjo_hbm.at[pl.ds(cur, _BLK)])
                pltpu.sync_copy(stg_f.at[pl.ds(b * _STG, _BLK)],
                                lno_hbm.at[pl.ds(cur, _BLK)])
                pltpu.sync_copy(stg_f.at[pl.ds((_NBUCK + b) * _STG, _BLK)],
                                fto_hbm.at[pl.ds(cur, _BLK)])

        fblk = _lane_val(bnd_v[pl.ds(32, 16)], 0)
        tblk = _lane_val(bnd_v[pl.ds(48, 16)], 0)
        tcnt = (tblk - wid + _NW - 1) // _NW

        def tbody(i, carry):
            off = (fblk + wid + i * _NW) * _BLK
            pltpu.sync_copy(dmp_i, tio_hbm.at[pl.ds(off, _BLK)])
            pltpu.sync_copy(dmp_i, tjo_hbm.at[pl.ds(off, _BLK)])
            pltpu.sync_copy(dmp_f, lno_hbm.at[pl.ds(off, _BLK)])
            pltpu.sync_copy(dmp_f, fto_hbm.at[pl.ds(off, _BLK)])
            return carry

        lax.fori_loop(0, tcnt, tbody, 0)

    return k(ti, tj, lens, feats, bases, bounds_flat)


def _sc_scatter_pair(msg, ti_p, bounds, k, e_rows):
    w = msg.shape[1]
    ablk = _ACCROWS // _BLK
    sz0 = min(_BSPAN, e_rows - 2 * k * _BSPAN) // _NS
    sz1 = min(_BSPAN, e_rows - (2 * k + 1) * _BSPAN) // _NS

    @functools.partial(
        pl.kernel,
        mesh=_sc_mesh(),
        out_type=jax.ShapeDtypeStruct((2 * _BSPAN, w), jnp.float32),
        scratch_types=[
            pltpu.VMEM((_BLK,), jnp.int32),
            pltpu.VMEM((_BLK,), jnp.int32),
            pltpu.VMEM((_BLK, w), jnp.float32),
            pltpu.VMEM((128,), jnp.int32),
            pltpu.VMEM_SHARED((_ACCROWS, w), jnp.float32),
            pltpu.SemaphoreType.DMA,
        ],
    )
    def kk(msg_hbm, tip_hbm, bd_hbm, out_hbm, tiv, tloc, rows_v,
           bnd_v, acc, sem):
        c = lax.axis_index("c")
        s = lax.axis_index("s")
        pltpu.sync_copy(bd_hbm, bnd_v)
        bs0 = _lane_val(bnd_v[pl.ds(0, 16)], 2 * k)
        bs1 = _lane_val(bnd_v[pl.ds(0, 16)], 2 * k + 1)
        be0 = _lane_val(bnd_v[pl.ds(16, 16)], 2 * k)
        be1 = _lane_val(bnd_v[pl.ds(16, 16)], 2 * k + 1)
        bs = jnp.where(c == 0, bs0, bs1)
        be = jnp.where(c == 0, be0, be1)

        def zrow(r, carry):
            for g in range(w // 16):
                rows_v[r, pl.ds(g * 16, 16)] = jnp.zeros((16,), jnp.float32)
            return carry

        lax.fori_loop(0, _BLK, zrow, 0)
        zcnt = (ablk - s + _NS - 1) // _NS

        def zblk(i, carry):
            pltpu.sync_copy(rows_v,
                            acc.at[pl.ds((s + i * _NS) * _BLK, _BLK)])
            return carry

        lax.fori_loop(0, zcnt, zblk, 0)
        plsc.subcore_barrier()

        bcnt = jnp.maximum(0, (be - bs - s + _NS - 1) // _NS)

        def body(i, carry):
            base = (bs + s + i * _NS) * _BLK
            pltpu.sync_copy(tip_hbm.at[pl.ds(base, _BLK)], tiv)
            for g in range(8):
                tloc[pl.ds(g * 16, 16)] = (
                    tiv[pl.ds(g * 16, 16)] - (2 * k + c) * _BSPAN)
            pltpu.sync_copy(msg_hbm.at[pl.ds(base, _BLK)], rows_v)
            pltpu.sync_copy(rows_v, acc.at[tloc], add=True)
            return carry

        lax.fori_loop(0, bcnt, body, 0)
        plsc.subcore_barrier()

        @pl.when(c == 0)
        def _():
            pltpu.sync_copy(acc.at[pl.ds(s * sz0, sz0)],
                            out_hbm.at[pl.ds(s * sz0, sz0)])

        @pl.when(c == 1)
        def _():
            pltpu.sync_copy(acc.at[pl.ds(s * sz1, sz1)],
                            out_hbm.at[pl.ds(_BSPAN + s * sz1, sz1)])

    return kk(msg, ti_p, bounds)


def _sc_scatter_big(msg, ti_p, bounds, e_rows):
    parts = []
    for k in range(_NBUCK // 2):
        pk = _sc_scatter_pair(msg, ti_p, bounds, k, e_rows)
        take = min(2 * _BSPAN, e_rows - 2 * k * _BSPAN)
        parts.append(pk if take == 2 * _BSPAN else pk[:take])
    return jnp.concatenate(parts, axis=0)


def kernel(atom_types, edge_index, bond_features, triple_bond_indices,
           triple_bond_lengths, triple_features, n_atoms, atom_embedding,
           W_bp, b_bp, W_up, b_up, W_3g, b_3g, W_3v, b_3v, W_ca, b_ca,
           W_ag, b_ag, W_av, b_av, W_ro, b_ro, W_f1, b_f1, W_f2, b_f2):
    n = atom_types.shape[0]
    e_cnt = edge_index.shape[1]
    u = atom_embedding.shape[1]
    g_cnt = n_atoms.shape[0]
    nblocks = W_up.shape[0]

    src = edge_index[0]
    dst = edge_index[1]
    ti = triple_bond_indices[:, 0]
    tj = triple_bond_indices[:, 1]

    x = _embed(atom_types, atom_embedding)
    e = _bond_proj(bond_features, W_bp, b_bp)

    for b in range(nblocks):
        e_t = _gather_rows(e, tj)
        msg = _gate(triple_bond_lengths, triple_features, e_t, W_up[b],
                    b_up[b], ch=2000)
        agg = _segsum(msg, ti, e_cnt)
        w1 = W_ca[b][:u]
        w2 = W_ca[b][u:2 * u]
        w3 = W_ca[b][2 * u:]
        xs, xd = _proj2(x, w1, w2)
        gs = _gather_rows(xs, src)
        gd = _gather_rows(xd, dst)
        e = _e_update(e, agg, gs, gd, W_3g[b], b_3g[b], W_3v[b], b_3v[b], w3,
                      b_ca[b])
        m2 = _sc_scatter_atoms(e, dst, n)
        np_ = m2.shape[0] // 2
        x = _x_update(x, m2[:n], m2[np_:np_ + n], W_ag[b], b_ag[b],
                      W_av[b], b_av[b])

    return _readout(x, g_cnt, W_ro, b_ro, W_f1, b_f1, W_f2, b_f2)
